# baseline (speedup 1.0000x reference)
import functools

import jax
import jax.numpy as jnp
from jax import lax
from jax.experimental import pallas as pl
from jax.experimental.pallas import tpu as pltpu
from jax.experimental.pallas import tpu_sc as plsc

DIM = 32
DIMW = DIM // 2
NC = 2
NS = 16
NW = NC * NS
GB = 32
NBUF = 2
CH = 625

_MESH = plsc.VectorSubcoreMesh(core_axis_name="c", subcore_axis_name="s")
_PARAMS = pltpu.CompilerParams(
    use_tc_tiling_on_sc=False, needs_layout_passes=False)


@functools.lru_cache(maxsize=None)
def _build_retype(vocab):
    rows_w = vocab // NW
    nch = rows_w // CH
    assert rows_w * NW == vocab and nch * CH == rows_w and nch % 2 == 0

    @functools.partial(
        pl.kernel,
        mesh=_MESH,
        out_type=jax.ShapeDtypeStruct((vocab, DIMW), jnp.int32),
        compiler_params=_PARAMS,
        scratch_types=[
            pltpu.VMEM((NBUF, CH, DIM), jnp.bfloat16),
            pltpu.VMEM((NBUF, CH, DIMW), jnp.int32),
            pltpu.SemaphoreType.DMA,
            pltpu.SemaphoreType.DMA,
            pltpu.SemaphoreType.DMA,
            pltpu.SemaphoreType.DMA,
        ],
    )
    def retype(w_hbm, out_hbm, bfv, i32v, si0, si1, so0, so1):
        sin = (si0, si1)
        sout = (so0, so1)
        wid = lax.axis_index("s") * NC + lax.axis_index("c")

        def base(c):
            return wid * rows_w + c * CH

        def in_copy(c, b):
            return pltpu.make_async_copy(
                w_hbm.at[pl.ds(base(c), CH)], bfv.at[b], sin[b])

        def out_copy(c, b):
            return pltpu.make_async_copy(
                i32v.at[b], out_hbm.at[pl.ds(base(c), CH)], sout[b])

        for b in range(NBUF):
            in_copy(b, b).start()

        def body(it, _):
            c0 = it * NBUF
            for b in range(NBUF):
                c = c0 + b
                in_copy(c, b).wait()

                @pl.when(c >= NBUF)
                def _():
                    out_copy(c - NBUF, b).wait()

                def rows(j, _):
                    for u in range(5):
                        r = j * 5 + u
                        i32v[b, r] = plsc.bitcast(bfv[b, r], jnp.int32)
                    return 0
                lax.fori_loop(0, CH // 5, rows, 0)

                @pl.when(c + NBUF < nch)
                def _():
                    in_copy(c + NBUF, b).start()

                out_copy(c, b).start()
            return 0

        lax.fori_loop(0, nch // NBUF, body, 0)
        for b in range(NBUF):
            out_copy(nch - NBUF + b, b).wait()

    return retype


@functools.lru_cache(maxsize=None)
def _build_gather(batch, hist):
    bat_w = batch // NW
    nblk = bat_w // GB
    assert bat_w * NW == batch and nblk * GB == bat_w and nblk % NBUF == 0

    @functools.partial(
        pl.kernel,
        mesh=_MESH,
        out_type=jax.ShapeDtypeStruct((hist, batch, DIM), jnp.bfloat16),
        compiler_params=_PARAMS,
        scratch_types=[
            pltpu.VMEM((bat_w, hist), jnp.int32),
            pltpu.VMEM((NBUF, GB, hist, DIMW), jnp.int32),
            pltpu.VMEM((hist, GB, DIM), jnp.bfloat16),
            pltpu.SemaphoreType.DMA,
            pltpu.SemaphoreType.DMA,
        ],
    )
    def emb(ids_hbm, t_hbm, out_hbm, idx_v, rows_v, bf_v, sem0, sem1):
        sems = (sem0, sem1)
        wid = lax.axis_index("s") * NC + lax.axis_index("c")
        b_base = wid * bat_w
        pltpu.sync_copy(ids_hbm.at[pl.ds(b_base, bat_w)], idx_v)

        def fire(blk_id, b):
            def one(j, _):
                pltpu.async_copy(
                    t_hbm.at[idx_v.at[blk_id * GB + j]],
                    rows_v.at[b].at[j],
                    sems[b],
                )
                return 0
            lax.fori_loop(0, GB, one, 0)

        def drain(blk_id, b):
            def one(j, _):
                pltpu.make_async_copy(
                    t_hbm.at[idx_v.at[blk_id * GB + j]],
                    rows_v.at[b].at[j],
                    sems[b],
                ).wait()
                return 0
            lax.fori_loop(0, GB, one, 0)

        def repack(b):
            def one(j, _):
                for r in range(hist):
                    w = rows_v[b, j, r]
                    lo = lax.bitwise_and(w, 0xFFFF)
                    hi = lax.shift_right_logical(w, 16)
                    p = plsc.pack(lo, hi, format=plsc.PackFormat.INTERLEAVED)
                    bf_v[r, j] = plsc.bitcast(p, jnp.bfloat16)
                return 0
            lax.fori_loop(0, GB, one, 0)

        for b in range(NBUF):
            fire(b, b)

        def body(it, _):
            g0 = it * NBUF
            for b in range(NBUF):
                blk_id = g0 + b
                drain(blk_id, b)
                repack(b)

                @pl.when(blk_id + NBUF < nblk)
                def _():
                    fire(blk_id + NBUF, b)

                pltpu.sync_copy(
                    bf_v, out_hbm.at[:, pl.ds(b_base + blk_id * GB, GB)]
                )
            return 0

        lax.fori_loop(0, nblk // NBUF, body, 0)

    return emb


def kernel(ids, weight):
    batch, hist = ids.shape
    table_i32 = _build_retype(weight.shape[0])(weight)
    out = _build_gather(batch, hist)(ids.astype(jnp.int32), table_i32)
    return jnp.swapaxes(out, 0, 1)


# speedup vs baseline: 1.0641x; 1.0641x over previous
import functools

import jax
import jax.numpy as jnp
from jax import lax
from jax.experimental import pallas as pl
from jax.experimental.pallas import tpu as pltpu
from jax.experimental.pallas import tpu_sc as plsc

DIM = 32
DIMW = DIM // 2
NC = 2
NS = 16
NW = NC * NS
GB = 32
NBUF = 2
CH = 625

_MESH = plsc.VectorSubcoreMesh(core_axis_name="c", subcore_axis_name="s")
_PARAMS = pltpu.CompilerParams(
    use_tc_tiling_on_sc=False, needs_layout_passes=False)


@functools.lru_cache(maxsize=None)
def _build_retype(vocab):
    rows_w = vocab // NW
    nch = rows_w // CH
    assert rows_w * NW == vocab and nch * CH == rows_w and nch % 2 == 0

    @functools.partial(
        pl.kernel,
        mesh=_MESH,
        out_type=jax.ShapeDtypeStruct((vocab, DIMW), jnp.int32),
        compiler_params=_PARAMS,
        scratch_types=[
            pltpu.VMEM((NBUF, CH, DIM), jnp.bfloat16),
            pltpu.VMEM((NBUF, CH, DIMW), jnp.int32),
            pltpu.SemaphoreType.DMA,
            pltpu.SemaphoreType.DMA,
            pltpu.SemaphoreType.DMA,
            pltpu.SemaphoreType.DMA,
        ],
    )
    def retype(w_hbm, out_hbm, bfv, i32v, si0, si1, so0, so1):
        sin = (si0, si1)
        sout = (so0, so1)
        wid = lax.axis_index("s") * NC + lax.axis_index("c")

        def base(c):
            return wid * rows_w + c * CH

        def in_copy(c, b):
            return pltpu.make_async_copy(
                w_hbm.at[pl.ds(base(c), CH)], bfv.at[b], sin[b])

        def out_copy(c, b):
            return pltpu.make_async_copy(
                i32v.at[b], out_hbm.at[pl.ds(base(c), CH)], sout[b])

        for b in range(NBUF):
            in_copy(b, b).start()

        def body(it, _):
            c0 = it * NBUF
            for b in range(NBUF):
                c = c0 + b
                in_copy(c, b).wait()

                @pl.when(c >= NBUF)
                def _():
                    out_copy(c - NBUF, b).wait()

                def rows(j, _):
                    for u in range(5):
                        r = j * 5 + u
                        i32v[b, r] = plsc.bitcast(bfv[b, r], jnp.int32)
                    return 0
                lax.fori_loop(0, CH // 5, rows, 0)

                @pl.when(c + NBUF < nch)
                def _():
                    in_copy(c + NBUF, b).start()

                out_copy(c, b).start()
            return 0

        lax.fori_loop(0, nch // NBUF, body, 0)
        for b in range(NBUF):
            out_copy(nch - NBUF + b, b).wait()

    return retype


@functools.lru_cache(maxsize=None)
def _build_gather(batch, hist):
    bat_w = batch // NW
    nblk = bat_w // GB
    assert bat_w * NW == batch and nblk * GB == bat_w and nblk % NBUF == 0

    @functools.partial(
        pl.kernel,
        mesh=_MESH,
        out_type=jax.ShapeDtypeStruct((batch, hist, DIM), jnp.bfloat16),
        compiler_params=_PARAMS,
        scratch_types=[
            pltpu.VMEM((bat_w, hist), jnp.int32),
            pltpu.VMEM((NBUF, GB, hist, DIMW), jnp.int32),
            pltpu.VMEM((GB, hist, DIM), jnp.bfloat16),
            pltpu.SemaphoreType.DMA,
            pltpu.SemaphoreType.DMA,
        ],
    )
    def emb(ids_hbm, t_hbm, out_hbm, idx_v, rows_v, bf_v, sem0, sem1):
        sems = (sem0, sem1)
        wid = lax.axis_index("s") * NC + lax.axis_index("c")
        b_base = wid * bat_w
        pltpu.sync_copy(ids_hbm.at[pl.ds(b_base, bat_w)], idx_v)

        def fire(blk_id, b):
            def one(j, _):
                pltpu.async_copy(
                    t_hbm.at[idx_v.at[blk_id * GB + j]],
                    rows_v.at[b].at[j],
                    sems[b],
                )
                return 0
            lax.fori_loop(0, GB, one, 0)

        def drain(blk_id, b):
            def one(j, _):
                pltpu.make_async_copy(
                    t_hbm.at[idx_v.at[blk_id * GB + j]],
                    rows_v.at[b].at[j],
                    sems[b],
                ).wait()
                return 0
            lax.fori_loop(0, GB, one, 0)

        def repack(b):
            def one(j, _):
                for r in range(hist):
                    w = rows_v[b, j, r]
                    lo = lax.bitwise_and(w, 0xFFFF)
                    hi = lax.shift_right_logical(w, 16)
                    p = plsc.pack(lo, hi, format=plsc.PackFormat.INTERLEAVED)
                    bf_v[j, r] = plsc.bitcast(p, jnp.bfloat16)
                return 0
            lax.fori_loop(0, GB, one, 0)

        for b in range(NBUF):
            fire(b, b)

        def body(it, _):
            g0 = it * NBUF
            for b in range(NBUF):
                blk_id = g0 + b
                drain(blk_id, b)
                repack(b)

                @pl.when(blk_id + NBUF < nblk)
                def _():
                    fire(blk_id + NBUF, b)

                pltpu.sync_copy(
                    bf_v, out_hbm.at[pl.ds(b_base + blk_id * GB, GB)]
                )
            return 0

        lax.fori_loop(0, nblk // NBUF, body, 0)

    return emb


def kernel(ids, weight):
    batch, hist = ids.shape
    table_i32 = _build_retype(weight.shape[0])(weight)
    return _build_gather(batch, hist)(ids.astype(jnp.int32), table_i32)
